# baseline (speedup 1.0000x reference)
import functools

import jax
import jax.numpy as jnp
from jax import lax
from jax.experimental import pallas as pl
from jax.experimental.pallas import tpu as pltpu
from jax.experimental.pallas import tpu_sc as plsc

N_MESH = 10000
N_GRID = 10000
E = 320000
D = 128
H = 128

NC = 2
NS = 16
NW = NC * NS
EPW = E // NW
K = 40
NCHUNK = EPW // K
NBUF = 5
KS = K
NCHUNK_S = EPW // KS
NBUF_S = 2


def _project_body(mesh_ref, grid_ref, w1b_ref, w1c_ref, mp_ref, gp_ref):
    mp_ref[...] = jnp.dot(mesh_ref[...], w1b_ref[...],
                          preferred_element_type=jnp.float32)
    gp_ref[...] = jnp.dot(grid_ref[...], w1c_ref[...],
                          preferred_element_type=jnp.float32)


def _project(mesh_nfeat, grid_nfeat, w1b, w1c):
    blk = 1000
    grid = (N_MESH // blk,)
    return pl.pallas_call(
        _project_body,
        grid=grid,
        in_specs=[
            pl.BlockSpec((blk, D), lambda i: (i, 0)),
            pl.BlockSpec((blk, D), lambda i: (i, 0)),
            pl.BlockSpec((D, H), lambda i: (0, 0)),
            pl.BlockSpec((D, H), lambda i: (0, 0)),
        ],
        out_specs=[
            pl.BlockSpec((blk, H), lambda i: (i, 0)),
            pl.BlockSpec((blk, H), lambda i: (i, 0)),
        ],
        out_shape=[
            jax.ShapeDtypeStruct((N_MESH, H), jnp.float32),
            jax.ShapeDtypeStruct((N_GRID, H), jnp.float32),
        ],
    )(mesh_nfeat, grid_nfeat, w1b, w1c)


def _gather_body(mp_hbm, gp_hbm, src_hbm, dst_hbm, mpg_hbm, gpg_hbm,
                 sidx_v, didx_v, *bufs):
    wid = lax.axis_index("c") * NS + lax.axis_index("s")
    base = wid * EPW
    rows_a = list(bufs[0:NBUF])
    rows_b = list(bufs[NBUF:2 * NBUF])
    gs_a = list(bufs[2 * NBUF:3 * NBUF])
    gs_b = list(bufs[3 * NBUF:4 * NBUF])
    ss_a = list(bufs[4 * NBUF:5 * NBUF])
    ss_b = list(bufs[5 * NBUF:6 * NBUF])
    pltpu.sync_copy(src_hbm.at[wid], sidx_v)
    pltpu.sync_copy(dst_hbm.at[wid], didx_v)

    ngrp = NCHUNK // NBUF

    def group(g, _):
        descs = {}

        for b in range(NBUF):
            j = NBUF * g + b

            @pl.when(g > 0)
            def _(b=b):
                pltpu.make_async_copy(rows_a[b],
                                      mpg_hbm.at[pl.ds(base, K)],
                                      ss_a[b]).wait()
                pltpu.make_async_copy(rows_b[b],
                                      gpg_hbm.at[pl.ds(base, K)],
                                      ss_b[b]).wait()

            descs[(b, 'a')] = pltpu.async_copy(
                mp_hbm.at[sidx_v.at[j]], rows_a[b], gs_a[b])
            descs[(b, 'b')] = pltpu.async_copy(
                gp_hbm.at[didx_v.at[j]], rows_b[b], gs_b[b])

        for b in range(NBUF):
            j = NBUF * g + b
            descs[(b, 'a')].wait()
            descs[(b, 'b')].wait()
            out_slc = pl.ds(pl.multiple_of(base + j * K, 8), K)
            pltpu.async_copy(rows_a[b], mpg_hbm.at[out_slc], ss_a[b])
            pltpu.async_copy(rows_b[b], gpg_hbm.at[out_slc], ss_b[b])
        return 0

    lax.fori_loop(0, ngrp, group, 0)
    for b in range(NBUF):
        pltpu.make_async_copy(rows_a[b], mpg_hbm.at[pl.ds(base, K)],
                              ss_a[b]).wait()
        pltpu.make_async_copy(rows_b[b], gpg_hbm.at[pl.ds(base, K)],
                              ss_b[b]).wait()


def _gather(mp, gp, src3d, dst3d):
    mesh = plsc.VectorSubcoreMesh(core_axis_name="c", subcore_axis_name="s")
    return pl.kernel(
        _gather_body,
        out_type=[
            jax.ShapeDtypeStruct((E, H), jnp.float32),
            jax.ShapeDtypeStruct((E, H), jnp.float32),
        ],
        mesh=mesh,
        scratch_types=(
            [pltpu.VMEM((NCHUNK, K), jnp.int32),
             pltpu.VMEM((NCHUNK, K), jnp.int32)]
            + [pltpu.VMEM((K, H), jnp.float32)] * (2 * NBUF)
            + [pltpu.SemaphoreType.DMA] * (4 * NBUF)
        ),
    )(mp, gp, src3d, dst3d)


def _edge_mlp_body(m2g_ref, mpg_ref, gpg_ref, w1a_ref, b1_ref, w2_ref,
                   b2_ref, g_ref, be_ref, out_ref):
    x = jnp.dot(m2g_ref[...].astype(jnp.bfloat16),
                w1a_ref[...].astype(jnp.bfloat16),
                preferred_element_type=jnp.float32)
    x = x + mpg_ref[...] + gpg_ref[...] + b1_ref[...]
    hact = x * jax.nn.sigmoid(x)
    y = jnp.dot(hact.astype(jnp.bfloat16), w2_ref[...].astype(jnp.bfloat16),
                preferred_element_type=jnp.float32)
    y = y + b2_ref[...]
    mu = jnp.mean(y, axis=-1, keepdims=True)
    var = jnp.mean(jnp.square(y - mu), axis=-1, keepdims=True)
    out_ref[...] = (y - mu) * lax.rsqrt(var + 1e-5) * g_ref[...] + be_ref[...]


def _edge_mlp(m2g, mpg, gpg, w1a, b1, w2, b2, g, be):
    blk = 2000
    grid = (E // blk,)
    vec = lambda i: (0, 0)
    return pl.pallas_call(
        _edge_mlp_body,
        grid=grid,
        in_specs=[
            pl.BlockSpec((blk, D), lambda i: (i, 0)),
            pl.BlockSpec((blk, H), lambda i: (i, 0)),
            pl.BlockSpec((blk, H), lambda i: (i, 0)),
            pl.BlockSpec((D, H), vec),
            pl.BlockSpec((1, H), vec),
            pl.BlockSpec((H, D), vec),
            pl.BlockSpec((1, D), vec),
            pl.BlockSpec((1, D), vec),
            pl.BlockSpec((1, D), vec),
        ],
        out_specs=pl.BlockSpec((blk, D), lambda i: (i, 0)),
        out_shape=jax.ShapeDtypeStruct((E, D), jnp.float32),
        compiler_params=pltpu.CompilerParams(
            dimension_semantics=("arbitrary",),
        ),
    )(m2g, mpg, gpg, w1a, b1, w2, b2, g, be)


def _scatter_body(efeat_hbm, dst3d_hbm, zeros_hbm, agg_hbm,
                  didx_v, *bufs):
    rows = list(bufs[0:NBUF_S])
    rsem = list(bufs[NBUF_S:2 * NBUF_S])
    agg_sh = bufs[2 * NBUF_S]
    cid = lax.axis_index("c")
    sid = lax.axis_index("s")
    wid = cid * NS + sid
    base = wid * EPW
    start = sid * 640
    @pl.when(sid < NS - 1)
    def _():
        pltpu.sync_copy(zeros_hbm.at[pl.ds(start, 640)],
                        agg_sh.at[pl.ds(start, 640)])

    @pl.when(sid == NS - 1)
    def _():
        pltpu.sync_copy(zeros_hbm.at[pl.ds(9600, 400)],
                        agg_sh.at[pl.ds(9600, 400)])

    pltpu.sync_copy(dst3d_hbm.at[wid], didx_v)
    plsc.subcore_barrier()

    ngrp = NCHUNK_S // NBUF_S

    def group(g, _):
        descs = {}
        for b in range(NBUF_S):
            j = NBUF_S * g + b
            descs[b] = pltpu.async_copy(
                efeat_hbm.at[pl.ds(pl.multiple_of(base + j * KS, 8), KS)],
                rows[b], rsem[b])
        for b in range(NBUF_S):
            j = NBUF_S * g + b
            descs[b].wait()
            pltpu.sync_copy(rows[b], agg_sh.at[didx_v.at[j]], add=True)
        return 0

    lax.fori_loop(0, ngrp, group, 0)
    plsc.subcore_barrier()

    @pl.when(sid < NS - 1)
    def _():
        pltpu.sync_copy(agg_sh.at[pl.ds(start, 640)],
                        agg_hbm.at[cid, pl.ds(start, 640)])

    @pl.when(sid == NS - 1)
    def _():
        pltpu.sync_copy(agg_sh.at[pl.ds(9600, 400)],
                        agg_hbm.at[cid, pl.ds(9600, 400)])


def _scatter(efeat, dst3d, zeros):
    mesh = plsc.VectorSubcoreMesh(core_axis_name="c", subcore_axis_name="s")
    return pl.kernel(
        _scatter_body,
        out_type=jax.ShapeDtypeStruct((NC, N_GRID, H), jnp.float32),
        mesh=mesh,
        scratch_types=(
            [pltpu.VMEM((NCHUNK_S, KS), jnp.int32)]
            + [pltpu.VMEM((KS, H), jnp.float32)] * NBUF_S
            + [pltpu.SemaphoreType.DMA] * NBUF_S
            + [pltpu.VMEM_SHARED((N_GRID, H), jnp.float32)]
        ),
    )(efeat, dst3d, zeros)


def _node_mlp_body(grid_ref, agg0_ref, agg1_ref, w1a_ref, w1b_ref, b1_ref,
                   w2_ref, b2_ref, g_ref, bn_ref, out_ref):
    agg = agg0_ref[...] + agg1_ref[...]
    x = jnp.dot(grid_ref[...].astype(jnp.bfloat16),
                w1a_ref[...].astype(jnp.bfloat16),
                preferred_element_type=jnp.float32)
    x = x + jnp.dot(agg.astype(jnp.bfloat16),
                    w1b_ref[...].astype(jnp.bfloat16),
                    preferred_element_type=jnp.float32)
    x = x + b1_ref[...]
    hact = x * jax.nn.sigmoid(x)
    y = jnp.dot(hact.astype(jnp.bfloat16), w2_ref[...].astype(jnp.bfloat16),
                preferred_element_type=jnp.float32)
    y = y + b2_ref[...]
    mu = jnp.mean(y, axis=-1, keepdims=True)
    var = jnp.mean(jnp.square(y - mu), axis=-1, keepdims=True)
    out_ref[...] = ((y - mu) * lax.rsqrt(var + 1e-5) * g_ref[...]
                    + bn_ref[...] + grid_ref[...])


def _node_mlp(grid_nfeat, agg0, agg1, w1a, w1b, b1, w2, b2, g, bn):
    blk = 1000
    grid = (N_GRID // blk,)
    vec = lambda i: (0, 0)
    return pl.pallas_call(
        _node_mlp_body,
        grid=grid,
        in_specs=[
            pl.BlockSpec((blk, D), lambda i: (i, 0)),
            pl.BlockSpec((blk, H), lambda i: (i, 0)),
            pl.BlockSpec((blk, H), lambda i: (i, 0)),
            pl.BlockSpec((D, H), vec),
            pl.BlockSpec((H, H), vec),
            pl.BlockSpec((1, H), vec),
            pl.BlockSpec((H, D), vec),
            pl.BlockSpec((1, D), vec),
            pl.BlockSpec((1, D), vec),
            pl.BlockSpec((1, D), vec),
        ],
        out_specs=pl.BlockSpec((blk, D), lambda i: (i, 0)),
        out_shape=jax.ShapeDtypeStruct((N_GRID, D), jnp.float32),
    )(grid_nfeat, agg0, agg1, w1a, w1b, b1, w2, b2, g, bn)


def kernel(m2g_efeat, grid_nfeat, mesh_nfeat, src_idx, dst_idx,
           W1e, b1e, W2e, b2e, gE, bE,
           W1n, b1n, W2n, b2n, gN, bN):
    w1a, w1b, w1c = W1e[:D], W1e[D:2 * D], W1e[2 * D:]

    mp, gp = _project(mesh_nfeat, grid_nfeat, w1b, w1c)
    mpg, gpg = _gather(mp, gp, src_idx.reshape(NW, NCHUNK, K),
                       dst_idx.reshape(NW, NCHUNK, K))
    efeat = _edge_mlp(m2g_efeat, mpg, gpg, w1a, b1e.reshape(1, H),
                      W2e, b2e.reshape(1, D), gE.reshape(1, D),
                      bE.reshape(1, D))
    zeros = jnp.zeros((N_GRID, H), jnp.float32)
    agg = _scatter(efeat, dst_idx.reshape(NW, NCHUNK_S, KS), zeros)
    out = _node_mlp(grid_nfeat, agg[0], agg[1],
                    W1n[:D], W1n[D:], b1n.reshape(1, H),
                    W2n, b2n.reshape(1, D), gN.reshape(1, D),
                    bN.reshape(1, D))
    return out


# speedup vs baseline: 1.0738x; 1.0738x over previous
import functools

import jax
import jax.numpy as jnp
from jax import lax
from jax.experimental import pallas as pl
from jax.experimental.pallas import tpu as pltpu
from jax.experimental.pallas import tpu_sc as plsc

N_MESH = 10000
N_GRID = 10000
E = 320000
D = 128
H = 128

NC = 2
NS = 16
NW = NC * NS
EPW = E // NW
K = 40
NCHUNK = EPW // K
NBUF = 5
KS = K
NCHUNK_S = EPW // KS
NBUF_S = 2


def _project_body(mesh_ref, grid_ref, w1b_ref, w1c_ref, mp_ref, gp_ref):
    mp_ref[...] = jnp.dot(mesh_ref[...], w1b_ref[...],
                          preferred_element_type=jnp.float32)
    gp_ref[...] = jnp.dot(grid_ref[...], w1c_ref[...],
                          preferred_element_type=jnp.float32)


def _project(mesh_nfeat, grid_nfeat, w1b, w1c):
    blk = 1000
    grid = (N_MESH // blk,)
    return pl.pallas_call(
        _project_body,
        grid=grid,
        in_specs=[
            pl.BlockSpec((blk, D), lambda i: (i, 0)),
            pl.BlockSpec((blk, D), lambda i: (i, 0)),
            pl.BlockSpec((D, H), lambda i: (0, 0)),
            pl.BlockSpec((D, H), lambda i: (0, 0)),
        ],
        out_specs=[
            pl.BlockSpec((blk, H), lambda i: (i, 0)),
            pl.BlockSpec((blk, H), lambda i: (i, 0)),
        ],
        out_shape=[
            jax.ShapeDtypeStruct((N_MESH, H), jnp.float32),
            jax.ShapeDtypeStruct((N_GRID, H), jnp.float32),
        ],
    )(mesh_nfeat, grid_nfeat, w1b, w1c)


def _gather_body(epw, mp_hbm, gp_hbm, src_hbm, dst_hbm, mpg_hbm, gpg_hbm,
                 sidx_v, didx_v, *bufs):
    nchunk = epw // K
    wid = lax.axis_index("c") * NS + lax.axis_index("s")
    base = wid * epw
    rows_a = list(bufs[0:NBUF])
    rows_b = list(bufs[NBUF:2 * NBUF])
    gs_a = list(bufs[2 * NBUF:3 * NBUF])
    gs_b = list(bufs[3 * NBUF:4 * NBUF])
    ss_a = list(bufs[4 * NBUF:5 * NBUF])
    ss_b = list(bufs[5 * NBUF:6 * NBUF])
    pltpu.sync_copy(src_hbm.at[wid], sidx_v)
    pltpu.sync_copy(dst_hbm.at[wid], didx_v)

    ngrp = nchunk // NBUF

    def group(g, _):
        descs = {}

        for b in range(NBUF):
            j = NBUF * g + b

            @pl.when(g > 0)
            def _(b=b):
                pltpu.make_async_copy(rows_a[b],
                                      mpg_hbm.at[pl.ds(base, K)],
                                      ss_a[b]).wait()
                pltpu.make_async_copy(rows_b[b],
                                      gpg_hbm.at[pl.ds(base, K)],
                                      ss_b[b]).wait()

            descs[(b, 'a')] = pltpu.async_copy(
                mp_hbm.at[sidx_v.at[j]], rows_a[b], gs_a[b])
            descs[(b, 'b')] = pltpu.async_copy(
                gp_hbm.at[didx_v.at[j]], rows_b[b], gs_b[b])

        for b in range(NBUF):
            j = NBUF * g + b
            descs[(b, 'a')].wait()
            descs[(b, 'b')].wait()
            out_slc = pl.ds(pl.multiple_of(base + j * K, 8), K)
            pltpu.async_copy(rows_a[b], mpg_hbm.at[out_slc], ss_a[b])
            pltpu.async_copy(rows_b[b], gpg_hbm.at[out_slc], ss_b[b])
        return 0

    lax.fori_loop(0, ngrp, group, 0)
    for b in range(NBUF):
        pltpu.make_async_copy(rows_a[b], mpg_hbm.at[pl.ds(base, K)],
                              ss_a[b]).wait()
        pltpu.make_async_copy(rows_b[b], gpg_hbm.at[pl.ds(base, K)],
                              ss_b[b]).wait()


def _gather(mp, gp, src3d, dst3d, n_edges):
    epw = n_edges // NW
    nchunk = epw // K
    mesh = plsc.VectorSubcoreMesh(core_axis_name="c", subcore_axis_name="s")
    return pl.kernel(
        functools.partial(_gather_body, epw),
        out_type=[
            jax.ShapeDtypeStruct((n_edges, H), jnp.float32),
            jax.ShapeDtypeStruct((n_edges, H), jnp.float32),
        ],
        mesh=mesh,
        scratch_types=(
            [pltpu.VMEM((nchunk, K), jnp.int32),
             pltpu.VMEM((nchunk, K), jnp.int32)]
            + [pltpu.VMEM((K, H), jnp.float32)] * (2 * NBUF)
            + [pltpu.SemaphoreType.DMA] * (4 * NBUF)
        ),
    )(mp, gp, src3d, dst3d)


def _edge_mlp_body(m2g_ref, mpg_ref, gpg_ref, w1a_ref, b1_ref, w2_ref,
                   b2_ref, g_ref, be_ref, out_ref):
    x = jnp.dot(m2g_ref[...].astype(jnp.bfloat16),
                w1a_ref[...].astype(jnp.bfloat16),
                preferred_element_type=jnp.float32)
    x = x + mpg_ref[...] + gpg_ref[...] + b1_ref[...]
    hact = x * jax.nn.sigmoid(x)
    y = jnp.dot(hact.astype(jnp.bfloat16), w2_ref[...].astype(jnp.bfloat16),
                preferred_element_type=jnp.float32)
    y = y + b2_ref[...]
    mu = jnp.mean(y, axis=-1, keepdims=True)
    var = jnp.mean(jnp.square(y - mu), axis=-1, keepdims=True)
    out_ref[...] = (y - mu) * lax.rsqrt(var + 1e-5) * g_ref[...] + be_ref[...]


def _edge_mlp(m2g, mpg, gpg, w1a, b1, w2, b2, g, be):
    blk = 2000
    n_edges = m2g.shape[0]
    grid = (n_edges // blk,)
    vec = lambda i: (0, 0)
    return pl.pallas_call(
        _edge_mlp_body,
        grid=grid,
        in_specs=[
            pl.BlockSpec((blk, D), lambda i: (i, 0)),
            pl.BlockSpec((blk, H), lambda i: (i, 0)),
            pl.BlockSpec((blk, H), lambda i: (i, 0)),
            pl.BlockSpec((D, H), vec),
            pl.BlockSpec((1, H), vec),
            pl.BlockSpec((H, D), vec),
            pl.BlockSpec((1, D), vec),
            pl.BlockSpec((1, D), vec),
            pl.BlockSpec((1, D), vec),
        ],
        out_specs=pl.BlockSpec((blk, D), lambda i: (i, 0)),
        out_shape=jax.ShapeDtypeStruct((n_edges, D), jnp.float32),
        compiler_params=pltpu.CompilerParams(
            dimension_semantics=("arbitrary",),
        ),
    )(m2g, mpg, gpg, w1a, b1, w2, b2, g, be)


def _scatter_body(epw, nbuf_s, efeat_hbm, dst3d_hbm, zeros_hbm, agg_hbm,
                  didx_v, *bufs):
    nchunk_s = epw // KS
    rows = list(bufs[0:nbuf_s])
    rsem = list(bufs[nbuf_s:2 * nbuf_s])
    agg_sh = bufs[2 * nbuf_s]
    cid = lax.axis_index("c")
    sid = lax.axis_index("s")
    wid = cid * NS + sid
    base = wid * epw
    start = sid * 640
    @pl.when(sid < NS - 1)
    def _():
        pltpu.sync_copy(zeros_hbm.at[pl.ds(start, 640)],
                        agg_sh.at[pl.ds(start, 640)])

    @pl.when(sid == NS - 1)
    def _():
        pltpu.sync_copy(zeros_hbm.at[pl.ds(9600, 400)],
                        agg_sh.at[pl.ds(9600, 400)])

    pltpu.sync_copy(dst3d_hbm.at[wid], didx_v)
    plsc.subcore_barrier()

    ngrp = nchunk_s // nbuf_s

    def group(g, _):
        descs = {}
        for b in range(nbuf_s):
            j = nbuf_s * g + b
            descs[b] = pltpu.async_copy(
                efeat_hbm.at[pl.ds(pl.multiple_of(base + j * KS, 8), KS)],
                rows[b], rsem[b])
        for b in range(nbuf_s):
            j = nbuf_s * g + b
            descs[b].wait()
            pltpu.sync_copy(rows[b], agg_sh.at[didx_v.at[j]], add=True)
        return 0

    lax.fori_loop(0, ngrp, group, 0)
    plsc.subcore_barrier()

    @pl.when(sid < NS - 1)
    def _():
        pltpu.sync_copy(agg_sh.at[pl.ds(start, 640)],
                        agg_hbm.at[cid, pl.ds(start, 640)])

    @pl.when(sid == NS - 1)
    def _():
        pltpu.sync_copy(agg_sh.at[pl.ds(9600, 400)],
                        agg_hbm.at[cid, pl.ds(9600, 400)])


def _scatter(efeat, dst3d, zeros, n_edges, nbuf_s):
    epw = n_edges // NW
    nchunk_s = epw // KS
    mesh = plsc.VectorSubcoreMesh(core_axis_name="c", subcore_axis_name="s")
    return pl.kernel(
        functools.partial(_scatter_body, epw, nbuf_s),
        out_type=jax.ShapeDtypeStruct((NC, N_GRID, H), jnp.float32),
        mesh=mesh,
        scratch_types=(
            [pltpu.VMEM((nchunk_s, KS), jnp.int32)]
            + [pltpu.VMEM((KS, H), jnp.float32)] * nbuf_s
            + [pltpu.SemaphoreType.DMA] * nbuf_s
            + [pltpu.VMEM_SHARED((N_GRID, H), jnp.float32)]
        ),
    )(efeat, dst3d, zeros)


def _node_mlp_body(grid_ref, agg0_ref, agg1_ref, agg2_ref, agg3_ref,
                   w1a_ref, w1b_ref, b1_ref,
                   w2_ref, b2_ref, g_ref, bn_ref, out_ref):
    agg = (agg0_ref[...] + agg1_ref[...]) + (agg2_ref[...] + agg3_ref[...])
    x = jnp.dot(grid_ref[...].astype(jnp.bfloat16),
                w1a_ref[...].astype(jnp.bfloat16),
                preferred_element_type=jnp.float32)
    x = x + jnp.dot(agg.astype(jnp.bfloat16),
                    w1b_ref[...].astype(jnp.bfloat16),
                    preferred_element_type=jnp.float32)
    x = x + b1_ref[...]
    hact = x * jax.nn.sigmoid(x)
    y = jnp.dot(hact.astype(jnp.bfloat16), w2_ref[...].astype(jnp.bfloat16),
                preferred_element_type=jnp.float32)
    y = y + b2_ref[...]
    mu = jnp.mean(y, axis=-1, keepdims=True)
    var = jnp.mean(jnp.square(y - mu), axis=-1, keepdims=True)
    out_ref[...] = ((y - mu) * lax.rsqrt(var + 1e-5) * g_ref[...]
                    + bn_ref[...] + grid_ref[...])


def _node_mlp(grid_nfeat, aggs, w1a, w1b, b1, w2, b2, g, bn):
    blk = 1000
    grid = (N_GRID // blk,)
    vec = lambda i: (0, 0)
    return pl.pallas_call(
        _node_mlp_body,
        grid=grid,
        in_specs=[
            pl.BlockSpec((blk, D), lambda i: (i, 0)),
            pl.BlockSpec((blk, H), lambda i: (i, 0)),
            pl.BlockSpec((blk, H), lambda i: (i, 0)),
            pl.BlockSpec((blk, H), lambda i: (i, 0)),
            pl.BlockSpec((blk, H), lambda i: (i, 0)),
            pl.BlockSpec((D, H), vec),
            pl.BlockSpec((H, H), vec),
            pl.BlockSpec((1, H), vec),
            pl.BlockSpec((H, D), vec),
            pl.BlockSpec((1, D), vec),
            pl.BlockSpec((1, D), vec),
            pl.BlockSpec((1, D), vec),
        ],
        out_specs=pl.BlockSpec((blk, D), lambda i: (i, 0)),
        out_shape=jax.ShapeDtypeStruct((N_GRID, D), jnp.float32),
    )(grid_nfeat, *aggs, w1a, w1b, b1, w2, b2, g, bn)


def kernel(m2g_efeat, grid_nfeat, mesh_nfeat, src_idx, dst_idx,
           W1e, b1e, W2e, b2e, gE, bE,
           W1n, b1n, W2n, b2n, gN, bN):
    w1a, w1b, w1c = W1e[:D], W1e[D:2 * D], W1e[2 * D:]

    mp, gp = _project(mesh_nfeat, grid_nfeat, w1b, w1c)
    zeros = jnp.zeros((N_GRID, H), jnp.float32)

    E2 = E // 2
    epw2 = E2 // NW
    nchunk2 = epw2 // K
    aggs = []
    for s in range(2):
        sl = slice(s * E2, (s + 1) * E2)
        src3d = src_idx[sl].reshape(NW, nchunk2, K)
        dst3d = dst_idx[sl].reshape(NW, nchunk2, K)
        mpg, gpg = _gather(mp, gp, src3d, dst3d, E2)
        efeat = _edge_mlp(m2g_efeat[sl], mpg, gpg, w1a, b1e.reshape(1, H),
                          W2e, b2e.reshape(1, D), gE.reshape(1, D),
                          bE.reshape(1, D))
        agg = _scatter(efeat, dst3d, zeros, E2, 5)
        aggs.extend([agg[0], agg[1]])

    out = _node_mlp(grid_nfeat, aggs,
                    W1n[:D], W1n[D:], b1n.reshape(1, H),
                    W2n, b2n.reshape(1, D), gN.reshape(1, D),
                    bN.reshape(1, D))
    return out
